# baseline (speedup 1.0000x reference)
import functools

import jax
import jax.numpy as jnp
import numpy as np
from jax.experimental import pallas as pl

_N = 2708
_F = 256
_K = 10
_E = 43328

def _make_r():
    skey = jax.random.key(42)
    return np.stack([
        np.asarray(jax.random.randint(jax.random.fold_in(skey, d),
                                      (_N, _K), 0, 2 ** 30,
                                      dtype=jnp.int32))
        for d in range(2)
    ])


_R_CONST = _make_r()


def _dense_body(nout, xc_ref, w_ref, b_ref, o_ref):
    z = jnp.dot(xc_ref[...], w_ref[...], preferred_element_type=jnp.float32)
    z = z + b_ref[...]
    h = 1.0 / (1.0 + jnp.exp(-z))
    col = jax.lax.broadcasted_iota(jnp.int32, h.shape, 1)
    hm = jnp.where(col < nout, h, 0.0)
    s = jnp.sum(hm * hm, axis=1, keepdims=True)
    o_ref[...] = hm * jax.lax.rsqrt(s)


def _dense(h_cat, W, b):
    m, kin = h_cat.shape
    nout = W.shape[0]
    mp = ((m + 7) // 8) * 8
    kp = ((kin + 127) // 128) * 128
    hp = jnp.pad(h_cat, ((0, mp - m), (0, kp - kin)))
    wp = jnp.pad(W.T, ((0, kp - kin), (0, 128 - nout)))
    bp = jnp.pad(b, (0, 128 - nout)).reshape(1, 128)
    out = pl.pallas_call(
        functools.partial(_dense_body, nout),
        out_shape=jax.ShapeDtypeStruct((mp, 128), jnp.float32),
    )(hp, wp, bp)
    return out[:m, :nout]


def kernel(x, edge_index, W1, b1, W2, b2):
    r_const = _R_CONST
    n = x.shape[0]
    src = jnp.concatenate([edge_index[0], edge_index[1]]).astype(jnp.int32)
    dst = jnp.concatenate([edge_index[1], edge_index[0]]).astype(jnp.int32)
    order = jnp.argsort(src)
    dst_sorted = dst[order]
    deg = jnp.bincount(src, length=n)
    start = jnp.concatenate([jnp.zeros((1,), deg.dtype), jnp.cumsum(deg)])[:-1]
    safe = jnp.maximum(deg, 1)

    h_old = x
    for d in range(2):
        r = jnp.asarray(r_const[d])
        idx = start[:, None] + (r % safe[:, None])
        nbrs = dst_sorted[idx]
        h_nbr = jnp.max(h_old[nbrs], axis=1)
        h_cat = jnp.concatenate([h_old, h_nbr], axis=1)
        if d == 0:
            h_old = _dense(h_cat, W1, b1)
        else:
            h_old = _dense(h_cat, W2, b2)
    return h_old


# speedup vs baseline: 1.8417x; 1.8417x over previous
import jax
import jax.numpy as jnp
import numpy as np
from jax import lax
from jax.experimental import pallas as pl
from jax.experimental.pallas import tpu as pltpu
from jax.experimental.pallas import tpu_sc as plsc

_N = 2708
_F = 256
_K = 10
_E = 43328
_2E = 2 * _E
_NP = 2816
_NT = 32
_NPT = _NP // _NT
_SPT = _NPT * _K
_SPAD = 896

_EPAD = 87040
_ECH = 2720
_EVEC = _ECH // 16
_NB = 4096
_NBV = _NB // 16
_NBT = _NB // _NT

_FCH = 4
_NSTEP = _NPT // _FCH


def _make_nodes():
    nodes = np.zeros((_NT, _SPAD), np.int32)
    sample_node = (np.arange(_NP * _K) // _K).reshape(_NT, _SPT)
    nodes[:, :_SPT] = sample_node
    return nodes


_NODES = _make_nodes()


def _make_r():
    skey = jax.random.key(42)
    rs = []
    for d in range(2):
        r = jax.random.randint(jax.random.fold_in(skey, d), (_N, _K),
                               0, 2 ** 30, dtype=jnp.int32)
        rp = jnp.pad(r, ((0, _NP - _N), (0, 0))).reshape(_NT, _SPT)
        rs.append(jnp.pad(rp, ((0, 0), (0, _SPAD - _SPT))))
    return jnp.stack(rs)


_MESH = plsc.VectorSubcoreMesh(core_axis_name="c", subcore_axis_name="s")
_SC_PARAMS = pltpu.CompilerParams(needs_layout_passes=False)


def _wid():
    return lax.axis_index("c") * 16 + lax.axis_index("s")


def _lane_shift(vtmp, vec, idxvec):
    vtmp[pl.ds(0, 16)] = vec
    return plsc.load_gather(vtmp, [idxvec])


def _seg_flags(vtmp, ss, iota):
    prev = _lane_shift(vtmp, ss, jnp.maximum(iota - 1, 0))
    nxt = _lane_shift(vtmp, ss, jnp.minimum(iota + 1, 15))
    first = jnp.logical_or(iota == 0, ss != prev)
    last = jnp.logical_or(iota == 15, ss != nxt)
    rr = iota - plsc.cummax(jnp.where(first, iota, 0))
    return last, rr


def _hist_body(ef_h, hist_h, srcb, histb, vtmp, sem):
    wid = _wid()
    pltpu.sync_copy(ef_h.at[pl.ds(wid * _ECH, _ECH)], srcb)
    iota = lax.iota(jnp.int32, 16)
    zero = jnp.zeros((16,), jnp.int32)

    def zb(i, _):
        histb[pl.ds(i * 16, 16)] = zero
        return _

    lax.fori_loop(0, _NBV, zb, None)

    def hb(v, _):
        s = srcb[pl.ds(v * 16, 16)]
        ks, _p = plsc.sort_key_val(s * 16 + iota, iota)
        ss = lax.shift_right_logical(ks, 4)
        last, rr = _seg_flags(vtmp, ss, iota)
        c0 = plsc.load_gather(histb, [ss])
        plsc.store_scatter(histb, [ss], c0 + rr + 1, mask=last)
        return _

    lax.fori_loop(0, _EVEC, hb, None)
    pltpu.sync_copy(histb, hist_h.at[wid])


def _sc_hist(ef_p):
    return pl.kernel(
        _hist_body,
        out_type=jax.ShapeDtypeStruct((_NT, _NB), jnp.int32),
        mesh=_MESH,
        compiler_params=_SC_PARAMS,
        scratch_types=[
            pltpu.VMEM((_ECH,), jnp.int32),
            pltpu.VMEM((_NB,), jnp.int32),
            pltpu.VMEM((16,), jnp.int32),
            pltpu.SemaphoreType.DMA,
        ],
    )(ef_p)


def _prefix_body(hist_h, base_h, deg_h, loc, out2, degl, sem):
    wid = _wid()
    pltpu.sync_copy(hist_h.at[:, pl.ds(wid * _NBT, _NBT)], loc)
    iota = lax.iota(jnp.int32, 16)

    for sl in range(_NBT // 16):
        cs = pl.ds(sl * 16, 16)
        acc = loc[0, cs]
        for c in range(1, _NT):
            acc = acc + loc[c, cs]
        degl[cs] = acc

    def pb(b, _):
        bvec = jnp.zeros((16,), jnp.int32) + b
        lo = plsc.load_gather(loc, [iota, bvec])
        hi = plsc.load_gather(loc, [iota + 16, bvec])
        lo_excl = plsc.cumsum(lo) - lo
        hi_excl = plsc.cumsum(hi) - hi + jnp.sum(lo)
        plsc.store_scatter(out2, [iota, bvec], lo_excl)
        plsc.store_scatter(out2, [iota + 16, bvec], hi_excl)
        return _

    lax.fori_loop(0, _NBT, pb, None)
    pltpu.sync_copy(out2, base_h.at[:, pl.ds(wid * _NBT, _NBT)])
    pltpu.sync_copy(degl, deg_h.at[pl.ds(wid * _NBT, _NBT)])


def _sc_prefix(hist):
    return pl.kernel(
        _prefix_body,
        out_type=(jax.ShapeDtypeStruct((_NT, _NB), jnp.int32),
                  jax.ShapeDtypeStruct((_NB,), jnp.int32)),
        mesh=_MESH,
        compiler_params=_SC_PARAMS,
        scratch_types=[
            pltpu.VMEM((_NT, _NBT), jnp.int32),
            pltpu.VMEM((_NT, _NBT), jnp.int32),
            pltpu.VMEM((_NBT,), jnp.int32),
            pltpu.SemaphoreType.DMA,
        ],
    )(hist)


def _rank_body(ef_h, dstf_h, base_h, deg_h,
               dsts_h, start_h,
               srcb, dstb, rowd, cur, start_t, posb, vtmp, sem):
    wid = _wid()
    pltpu.sync_copy(ef_h.at[pl.ds(wid * _ECH, _ECH)], srcb)
    pltpu.sync_copy(dstf_h.at[pl.ds(wid * _ECH, _ECH)], dstb)
    pltpu.sync_copy(base_h.at[wid], cur)
    pltpu.sync_copy(deg_h, rowd)
    iota = lax.iota(jnp.int32, 16)

    def scan_b(i, carry):
        sl = pl.ds(i * 16, 16)
        v = rowd[sl]
        st = plsc.cumsum(v) - v + carry
        start_t[sl] = st
        cur[sl] = cur[sl] + st
        return carry + jnp.sum(v)

    lax.fori_loop(0, _NBV, scan_b, jnp.int32(0))

    def rb(v, _):
        s = srcb[pl.ds(v * 16, 16)]
        ks, perm = plsc.sort_key_val(s * 16 + iota, iota)
        ss = lax.shift_right_logical(ks, 4)
        last, rr = _seg_flags(vtmp, ss, iota)
        c0 = plsc.load_gather(cur, [ss])
        plsc.store_scatter(cur, [ss], c0 + rr + 1, mask=last)
        plsc.store_scatter(posb, [perm + v * 16], c0 + rr)
        return _

    lax.fori_loop(0, _EVEC, rb, None)

    copies = [
        pltpu.make_async_copy(dstb.at[pl.ds(o, sz)],
                              dsts_h.at[posb.at[pl.ds(o, sz)]], sem)
        for o, sz in [(jb * 128, 128) for jb in range(_ECH // 128)]
        + [(_ECH - _ECH % 128, _ECH % 128)]
    ]
    for c in copies:
        c.start()
    for c in copies:
        c.wait()

    @pl.when(wid == 0)
    def _():
        pltpu.sync_copy(start_t, start_h)


def _sc_rank_scatter(ef_p, dstf_p, base_all, deg):
    return pl.kernel(
        _rank_body,
        out_type=(jax.ShapeDtypeStruct((_EPAD,), jnp.int32),
                  jax.ShapeDtypeStruct((_NB,), jnp.int32)),
        mesh=_MESH,
        compiler_params=_SC_PARAMS,
        scratch_types=[
            pltpu.VMEM((_ECH,), jnp.int32),
            pltpu.VMEM((_ECH,), jnp.int32),
            pltpu.VMEM((_NB,), jnp.int32),
            pltpu.VMEM((_NB,), jnp.int32),
            pltpu.VMEM((_NB,), jnp.int32),
            pltpu.VMEM((_ECH,), jnp.int32),
            pltpu.VMEM((16,), jnp.int32),
            pltpu.SemaphoreType.DMA,
        ],
    )(ef_p, dstf_p, base_all, deg)


def _sample_agg0_body(dsts_h, deg_h, start_h, x_h, nodes_h, r_h,
                      hnbr0, nbrs1_h,
                      deg_t, start_t, nodes_t, r_t, idxb,
                      nbr0, nbr1, fba, fbb, obuf, semg, sema, semb):
    wid = _wid()
    pltpu.sync_copy(deg_h, deg_t)
    pltpu.sync_copy(start_h, start_t)
    pltpu.sync_copy(nodes_h.at[wid], nodes_t)
    pltpu.sync_copy(r_h.at[0].at[wid], r_t.at[0])
    pltpu.sync_copy(r_h.at[1].at[wid], r_t.at[1])
    zero = jnp.zeros((16,), jnp.int32)

    for d in range(2):
        nbrbuf = nbr0 if d == 0 else nbr1

        def samp_body(v, _):
            sl = pl.ds(v * 16, 16)
            nd = nodes_t[sl]
            dg = plsc.load_gather(deg_t, [nd])
            st = plsc.load_gather(start_t, [nd])
            m = lax.rem(r_t[d, sl], jnp.maximum(dg, 1))
            idxb[sl] = jnp.minimum(st + m, _2E - 1)
            return _

        lax.fori_loop(0, _SPAD // 16, samp_body, None)

        copies = [
            pltpu.make_async_copy(dsts_h.at[idxb.at[pl.ds(jb * 128, 128)]],
                                  nbrbuf.at[pl.ds(jb * 128, 128)], semg)
            for jb in range(_SPAD // 128)
        ]
        for c in copies:
            c.start()
        for c in copies:
            c.wait()

    pltpu.sync_copy(nbr1, nbrs1_h.at[wid])

    for i in range(_SPAD // 16, 1024 // 16):
        nbr0[pl.ds(i * 16, 16)] = zero

    def _issue(off, fb, sem):
        return pltpu.make_async_copy(
            x_h.at[nbr0.at[pl.ds(off, _FCH * _K)]], fb, sem)

    def _wait(fb, sem):
        pltpu.make_async_copy(x_h.at[pl.ds(0, _FCH * _K)], fb, sem).wait()

    def _compute(fb, node_off):
        for n in range(_FCH):
            for q in range(_F // 16):
                cs = pl.ds(q * 16, 16)
                acc = fb[n * _K, cs]
                for j in range(1, _K):
                    acc = jnp.maximum(acc, fb[n * _K + j, cs])
                obuf[n, cs] = acc
        pltpu.sync_copy(obuf, hnbr0.at[pl.ds(node_off, _FCH)])

    _issue(0, fba, sema).start()

    def agg_pair(g, _):
        _issue(g * 80 + 40, fbb, semb).start()
        _wait(fba, sema)
        _compute(fba, wid * _NPT + g * 8)
        _issue(g * 80 + 80, fba, sema).start()
        _wait(fbb, semb)
        _compute(fbb, wid * _NPT + g * 8 + 4)
        return _

    lax.fori_loop(0, _NSTEP // 2, agg_pair, None)
    _wait(fba, sema)


def _sc_sample_agg0(dsts, deg, start, xp, nodes_c, r_in):
    return pl.kernel(
        _sample_agg0_body,
        out_type=(jax.ShapeDtypeStruct((_NP, _F), jnp.float32),
                  jax.ShapeDtypeStruct((_NT, _SPAD), jnp.int32)),
        mesh=_MESH,
        compiler_params=_SC_PARAMS,
        scratch_types=[
            pltpu.VMEM((_NB,), jnp.int32),
            pltpu.VMEM((_NB,), jnp.int32),
            pltpu.VMEM((_SPAD,), jnp.int32),
            pltpu.VMEM((2, _SPAD), jnp.int32),
            pltpu.VMEM((_SPAD,), jnp.int32),
            pltpu.VMEM((1024,), jnp.int32),
            pltpu.VMEM((_SPAD,), jnp.int32),
            pltpu.VMEM((_FCH * _K, _F), jnp.float32),
            pltpu.VMEM((_FCH * _K, _F), jnp.float32),
            pltpu.VMEM((_FCH, _F), jnp.float32),
            pltpu.SemaphoreType.DMA,
            pltpu.SemaphoreType.DMA,
            pltpu.SemaphoreType.DMA,
        ],
    )(dsts, deg, start, xp, nodes_c, r_in)


def _agg1_body(h_h, nbrs1_h, hnbr1, h0loc, nbr1, obuf, sem):
    wid = _wid()
    pltpu.sync_copy(h_h, h0loc)
    pltpu.sync_copy(nbrs1_h.at[wid], nbr1)
    iota = lax.iota(jnp.int32, 16)

    def node_body(n, _):
        nv = plsc.load_gather(nbr1, [jnp.minimum(n * _K + iota, _SPAD - 1)])
        rows = [nv[j] * 32 for j in range(_K)]
        for q in range(2):
            acc = h0loc[pl.ds(rows[0] + q * 16, 16)]
            for j in range(1, _K):
                acc = jnp.maximum(acc, h0loc[pl.ds(rows[j] + q * 16, 16)])
            obuf[pl.ds(n * 32 + q * 16, 16)] = acc
        return _

    lax.fori_loop(0, _NPT, node_body, None)
    pltpu.sync_copy(obuf, hnbr1.at[pl.ds(wid * _NPT * 32, _NPT * 32)])


def _sc_agg1(h0flat, nbrs1):
    return pl.kernel(
        _agg1_body,
        out_type=jax.ShapeDtypeStruct((_NP * 32,), jnp.float32),
        mesh=_MESH,
        compiler_params=_SC_PARAMS,
        scratch_types=[
            pltpu.VMEM((_NP * 32,), jnp.float32),
            pltpu.VMEM((_SPAD,), jnp.int32),
            pltpu.VMEM((_NPT * 32,), jnp.float32),
            pltpu.SemaphoreType.DMA,
        ],
    )(h0flat, nbrs1)


def _dense0_body(x_ref, h_ref, wa_ref, wb_ref, b_ref, o_ref):
    z = jnp.dot(x_ref[...], wa_ref[...], preferred_element_type=jnp.float32)
    z = z + jnp.dot(h_ref[...], wb_ref[...], preferred_element_type=jnp.float32)
    z = z + b_ref[...]
    h = 1.0 / (1.0 + jnp.exp(-z))
    col = lax.broadcasted_iota(jnp.int32, h.shape, 1)
    hm = jnp.where(col < 20, h, 0.0)
    s = jnp.sum(hm * hm, axis=1, keepdims=True)
    o_ref[...] = hm * lax.rsqrt(s)


def _dense1_body(h_ref, hn_ref, wa_ref, wb_ref, b_ref, o_ref):
    z = jnp.dot(h_ref[...], wa_ref[...], preferred_element_type=jnp.float32)
    z = z + jnp.dot(hn_ref[...], wb_ref[...], preferred_element_type=jnp.float32)
    z = z + b_ref[...]
    h = 1.0 / (1.0 + jnp.exp(-z))
    col = lax.broadcasted_iota(jnp.int32, h.shape, 1)
    hm = jnp.where(col < 10, h, 0.0)
    s = jnp.sum(hm * hm, axis=1, keepdims=True)
    o_ref[...] = hm * lax.rsqrt(s)


def kernel(x, edge_index, W1, b1, W2, b2):
    ef = edge_index.reshape(-1).astype(jnp.int32)
    dstf = edge_index[::-1].reshape(-1).astype(jnp.int32)
    ef_p = jnp.pad(ef, (0, _EPAD - _2E), constant_values=_NP)
    dstf_p = jnp.pad(dstf, (0, _EPAD - _2E))
    xp = jnp.pad(x, ((0, _NP - _N), (0, 0)))
    nodes_c = jnp.asarray(_NODES)
    r_in = _make_r()

    hist = _sc_hist(ef_p)
    base_all, deg = _sc_prefix(hist)
    dst_sorted, start = _sc_rank_scatter(ef_p, dstf_p, base_all, deg)

    hnbr0, nbrs1 = _sc_sample_agg0(dst_sorted, deg, start, xp,
                                   nodes_c, r_in)

    w1t = jnp.pad(W1.T, ((0, 0), (0, 32 - 20)))
    b1p = jnp.pad(b1, (0, 32 - 20)).reshape(1, 32)
    h0 = pl.pallas_call(
        _dense0_body,
        out_shape=jax.ShapeDtypeStruct((_NP, 32), jnp.float32),
    )(xp, hnbr0, w1t[:_F], w1t[_F:], b1p)

    hnbr1 = _sc_agg1(h0.reshape(-1), nbrs1).reshape(_NP, 32)

    w2a = jnp.zeros((32, 128), jnp.float32).at[0:20, 0:10].set(W2.T[0:20])
    w2b = jnp.zeros((32, 128), jnp.float32).at[0:20, 0:10].set(W2.T[20:40])
    b2p = jnp.pad(b2, (0, 128 - 10)).reshape(1, 128)
    out = pl.pallas_call(
        _dense1_body,
        out_shape=jax.ShapeDtypeStruct((_NP, 128), jnp.float32),
    )(h0, hnbr1, w2a, w2b, b2p)
    return out[:_N, :10]
